# Initial kernel scaffold; baseline (speedup 1.0000x reference)
import jax
import jax.numpy as jnp
from jax.experimental import pallas as pl


def kernel(feat, coord, batch, params):
    raise NotImplementedError("write your pallas kernel here")


# speedup vs baseline: 1.1125x; 1.1125x over previous
import jax

jax.config.update("jax_enable_x64", True)

import jax.numpy as jnp
import numpy as np
from jax.experimental import pallas as pl

_GRID = 0.02
_PATCH = 1024
_ENC_CHANNELS = (32, 64, 128, 256)


def _ln(x):
    m = jnp.mean(x, axis=-1, keepdims=True)
    v = jnp.var(x, axis=-1, keepdims=True)
    return (x - m) / jnp.sqrt(v + 1e-5)


def _attn(x, Wqkv, Wo, heads, patch):
    N, C = x.shape
    P = min(patch, N)
    hd = C // heads
    qkv = (x @ Wqkv).reshape(N // P, P, 3, heads, hd)
    q = jnp.transpose(qkv[:, :, 0], (0, 2, 1, 3))
    k = jnp.transpose(qkv[:, :, 1], (0, 2, 1, 3))
    v = jnp.transpose(qkv[:, :, 2], (0, 2, 1, 3))
    s = (q @ jnp.swapaxes(k, -1, -2)) / np.float32(np.sqrt(hd))
    a = jax.nn.softmax(s, axis=-1)
    o = a @ v
    o = jnp.transpose(o, (0, 2, 1, 3)).reshape(N, C)
    return o @ Wo


def _block(x, p, heads):
    x = x + _attn(_ln(x), p['Wqkv'], p['Wo'], heads, _PATCH)
    x = x + jax.nn.gelu(_ln(x) @ p['W1']) @ p['W2']
    return x


def kernel(feat, coord, batch, params):
    n = feat.shape[0]
    g = jnp.clip(jnp.floor(coord / _GRID).astype(jnp.int32), 0, 1023)
    code = jnp.zeros((n,), dtype=jnp.int32)
    for b in range(10):
        for a in range(3):
            code = code | (((g[:, a] >> b) & 1) << (3 * b + a))
    iota = jnp.arange(n, dtype=jnp.int32)
    _, _, order = jax.lax.sort(
        (batch.astype(jnp.int32), code, iota),
        dimension=0, is_stable=True, num_keys=2)
    x = feat[order] @ params['W_embed'] + params['b_embed']
    blk = 0
    for s in range(4):
        C = _ENC_CHANNELS[s]
        if s > 0:
            x = x @ params['pool_W'][s - 1]
            x = x.reshape(x.shape[0] // 2, 2, C).max(axis=1)
        x = _block(x, params['blocks'][blk], C // 32)
        blk += 1
    x = _block(x, params['blocks'][blk], _ENC_CHANNELS[-1] // 32)
    return x


# speedup vs baseline: 2.1978x; 1.9755x over previous
import functools

import jax

jax.config.update("jax_enable_x64", True)

import jax.numpy as jnp
import numpy as np
from jax.experimental import pallas as pl
from jax.experimental.pallas import tpu as pltpu

_GRID = 0.02
_P = 1024
_ENC_CHANNELS = (32, 64, 128, 256)


def _ln(x):
    m = jnp.mean(x, axis=-1, keepdims=True)
    v = jnp.var(x, axis=-1, keepdims=True)
    return (x - m) / jnp.sqrt(v + 1e-5)


def _block_body(x, wqkv, wo, w1, w2, heads):
    C = x.shape[-1]
    hd = C // heads
    xl = _ln(x)
    qkv = jnp.dot(xl, wqkv, preferred_element_type=jnp.float32)
    scale = np.float32(1.0 / np.sqrt(hd))
    outs = []
    for h in range(heads):
        q = qkv[:, h * hd:(h + 1) * hd]
        k = qkv[:, C + h * hd:C + (h + 1) * hd]
        v = qkv[:, 2 * C + h * hd:2 * C + (h + 1) * hd]
        s = jax.lax.dot_general(q, k, (((1,), (1,)), ((), ())),
                                preferred_element_type=jnp.float32) * scale
        s = s - jnp.max(s, axis=-1, keepdims=True)
        e = jnp.exp(s)
        a = e / jnp.sum(e, axis=-1, keepdims=True)
        outs.append(jnp.dot(a, v, preferred_element_type=jnp.float32))
    o = outs[0] if heads == 1 else jnp.concatenate(outs, axis=-1)
    x = x + jnp.dot(o, wo, preferred_element_type=jnp.float32)
    xl2 = _ln(x)
    hmid = jax.nn.gelu(jnp.dot(xl2, w1, preferred_element_type=jnp.float32))
    return x + jnp.dot(hmid, w2, preferred_element_type=jnp.float32)


def _embed_block_kernel(fg_ref, we_ref, be_ref, wqkv_ref, wo_ref, w1_ref,
                        w2_ref, o_ref, *, heads):
    x = jnp.dot(fg_ref[...], we_ref[...],
                preferred_element_type=jnp.float32) + be_ref[...]
    o_ref[...] = _block_body(x, wqkv_ref[...], wo_ref[...], w1_ref[...],
                             w2_ref[...], heads)


def _pool_block_kernel(xp_ref, pw_ref, wqkv_ref, wo_ref, w1_ref, w2_ref,
                       o_ref, *, heads):
    a = jnp.dot(xp_ref[:, 0, :], pw_ref[...],
                preferred_element_type=jnp.float32)
    b = jnp.dot(xp_ref[:, 1, :], pw_ref[...],
                preferred_element_type=jnp.float32)
    x = jnp.maximum(a, b)
    o_ref[...] = _block_body(x, wqkv_ref[...], wo_ref[...], w1_ref[...],
                             w2_ref[...], heads)


def _plain_block_kernel(x_ref, wqkv_ref, wo_ref, w1_ref, w2_ref, o_ref, *,
                        heads):
    o_ref[...] = _block_body(x_ref[...], wqkv_ref[...], wo_ref[...],
                             w1_ref[...], w2_ref[...], heads)


_Z = lambda: jnp.int32(0)


def _full_spec(shape):
    nd = len(shape)
    return pl.BlockSpec(shape, lambda w: (_Z(),) * nd)


_CPARAMS = pltpu.CompilerParams(
    dimension_semantics=("arbitrary",),
)


def _embed_block_call(fg, we, be, blk, heads):
    n = fg.shape[0]
    C = we.shape[1]
    grid = (n // _P,)
    return pl.pallas_call(
        functools.partial(_embed_block_kernel, heads=heads),
        grid=grid,
        in_specs=[
            pl.BlockSpec((_P, fg.shape[1]), lambda w: (w, _Z())),
            _full_spec(we.shape),
            _full_spec((1, C)),
            _full_spec(blk['Wqkv'].shape),
            _full_spec(blk['Wo'].shape),
            _full_spec(blk['W1'].shape),
            _full_spec(blk['W2'].shape),
        ],
        out_specs=pl.BlockSpec((_P, C), lambda w: (w, _Z())),
        out_shape=jax.ShapeDtypeStruct((n, C), jnp.float32),
        compiler_params=_CPARAMS,
    )(fg, we, be.reshape(1, C), blk['Wqkv'], blk['Wo'], blk['W1'], blk['W2'])


def _pool_block_call(x, pw, blk, heads):
    n2 = x.shape[0] // 2
    Cp = x.shape[1]
    C = pw.shape[1]
    xp = x.reshape(n2, 2, Cp)
    grid = (n2 // _P,)
    return pl.pallas_call(
        functools.partial(_pool_block_kernel, heads=heads),
        grid=grid,
        in_specs=[
            pl.BlockSpec((_P, 2, Cp), lambda w: (w, _Z(), _Z())),
            _full_spec(pw.shape),
            _full_spec(blk['Wqkv'].shape),
            _full_spec(blk['Wo'].shape),
            _full_spec(blk['W1'].shape),
            _full_spec(blk['W2'].shape),
        ],
        out_specs=pl.BlockSpec((_P, C), lambda w: (w, _Z())),
        out_shape=jax.ShapeDtypeStruct((n2, C), jnp.float32),
        compiler_params=_CPARAMS,
    )(xp, pw, blk['Wqkv'], blk['Wo'], blk['W1'], blk['W2'])


def _plain_block_call(x, blk, heads):
    n, C = x.shape
    grid = (n // _P,)
    return pl.pallas_call(
        functools.partial(_plain_block_kernel, heads=heads),
        grid=grid,
        in_specs=[
            pl.BlockSpec((_P, C), lambda w: (w, _Z())),
            _full_spec(blk['Wqkv'].shape),
            _full_spec(blk['Wo'].shape),
            _full_spec(blk['W1'].shape),
            _full_spec(blk['W2'].shape),
        ],
        out_specs=pl.BlockSpec((_P, C), lambda w: (w, _Z())),
        out_shape=jax.ShapeDtypeStruct((n, C), jnp.float32),
        compiler_params=_CPARAMS,
    )(x, blk['Wqkv'], blk['Wo'], blk['W1'], blk['W2'])


def kernel(feat, coord, batch, params):
    n = feat.shape[0]
    g = jnp.clip(jnp.floor(coord / _GRID).astype(jnp.int32), 0, 1023)
    code = jnp.zeros((n,), dtype=jnp.int32)
    for b in range(10):
        for a in range(3):
            code = code | (((g[:, a] >> b) & 1) << (3 * b + a))
    iota = jnp.arange(n, dtype=jnp.int32)
    _, _, order = jax.lax.sort(
        (batch.astype(jnp.int32), code, iota),
        dimension=0, is_stable=True, num_keys=2)
    fg = feat[order]

    p = params
    x = _embed_block_call(fg, p['W_embed'], p['b_embed'], p['blocks'][0],
                          _ENC_CHANNELS[0] // 32)
    for s in range(1, 4):
        C = _ENC_CHANNELS[s]
        x = _pool_block_call(x, p['pool_W'][s - 1], p['blocks'][s], C // 32)
    x = _plain_block_call(x, p['blocks'][4], _ENC_CHANNELS[-1] // 32)
    return x
